# baseline (speedup 1.0000x reference)
import functools

import jax
import jax.numpy as jnp
from jax import lax
from jax.experimental import pallas as pl
from jax.experimental.pallas import tpu as pltpu

N = 4096
D_IN = 768
D_Z = 32
K = 8192
BR = 256
BETA = 0.25


def _dot_nt(a, b):
    return lax.dot_general(a, b, (((1,), (1,)), ((), ())),
                           preferred_element_type=jnp.float32)


def _dot_nn_exact(a, b):
    return lax.dot_general(a, b, (((1,), (0,)), ((), ())),
                           preferred_element_type=jnp.float32,
                           precision=lax.Precision.HIGHEST)


def _body(x_ref,
          eW0, eW1, eW2, eW3, eb0, eb1, eb2, eb3,
          dW0, dW1, dW2, dW3, db0, db1, db2, db3,
          cb0_ref, cb1_ref, cb2_ref,
          out_ref, idx_ref, loss_ref):
    h = x_ref[...]
    enc = ((eW0, eb0), (eW1, eb1), (eW2, eb2), (eW3, eb3))
    for i, (w, b) in enumerate(enc):
        h = _dot_nt(h, w[...]) + b[...]
        if i != 3:
            h = jnp.maximum(h, 0.0)
    z = h

    res = z
    xq_acc = jnp.zeros_like(z)
    total = jnp.float32(0.0)
    idx_cols = []
    for cb_ref in (cb0_ref, cb1_ref, cb2_ref):
        cb = cb_ref[...]
        zsq = jnp.sum(res * res, axis=1, keepdims=True)
        cbsq = jnp.sum(cb * cb, axis=1)[None, :]
        d = (zsq + cbsq) - 2.0 * _dot_nt(res, cb)
        dmin = jnp.min(d, axis=1, keepdims=True)
        iota = lax.broadcasted_iota(jnp.int32, (BR, K), 1)
        idx = jnp.min(jnp.where(d == dmin, iota, K), axis=1)
        onehot = (iota == idx[:, None]).astype(jnp.float32)
        xq = _dot_nn_exact(onehot, cb)
        diff = xq - res
        total = total + jnp.sum(diff * diff)
        res = res - xq
        xq_acc = xq_acc + xq
        idx_cols.append(idx[:, None])
    idx_ref[...] = jnp.concatenate(idx_cols, axis=1)

    h = xq_acc
    dec = ((dW0, db0), (dW1, db1), (dW2, db2), (dW3, db3))
    for i, (w, b) in enumerate(dec):
        h = _dot_nt(h, w[...]) + b[...]
        if i != 3:
            h = jnp.maximum(h, 0.0)
    out_ref[...] = h

    @pl.when(pl.program_id(0) == 0)
    def _():
        loss_ref[0, 0] = 0.0

    loss_ref[0, 0] += total


def kernel(x, enc_W0, enc_b0, enc_W1, enc_b1, enc_W2, enc_b2, enc_W3,
           enc_b3, dec_W0, dec_b0, dec_W1, dec_b1, dec_W2, dec_b2, dec_W3,
           dec_b3, cb0, cb1, cb2):
    enc_Ws = (enc_W0, enc_W1, enc_W2, enc_W3)
    enc_bs = tuple(b[None, :] for b in (enc_b0, enc_b1, enc_b2, enc_b3))
    dec_Ws = (dec_W0, dec_W1, dec_W2, dec_W3)
    dec_bs = tuple(b[None, :] for b in (dec_b0, dec_b1, dec_b2, dec_b3))

    full = lambda s: pl.BlockSpec(s, lambda i: (0,) * len(s))
    in_specs = (
        [pl.BlockSpec((BR, D_IN), lambda i: (i, 0))]
        + [full(w.shape) for w in enc_Ws]
        + [full(b.shape) for b in enc_bs]
        + [full(w.shape) for w in dec_Ws]
        + [full(b.shape) for b in dec_bs]
        + [full((K, D_Z))] * 3
    )
    out, idx, loss = pl.pallas_call(
        _body,
        grid=(N // BR,),
        in_specs=in_specs,
        out_specs=(
            pl.BlockSpec((BR, D_IN), lambda i: (i, 0)),
            pl.BlockSpec((BR, 3), lambda i: (i, 0)),
            pl.BlockSpec(memory_space=pltpu.SMEM),
        ),
        out_shape=(
            jax.ShapeDtypeStruct((N, D_IN), jnp.float32),
            jax.ShapeDtypeStruct((N, 3), jnp.int32),
            jax.ShapeDtypeStruct((1, 1), jnp.float32),
        ),
    )(x, *enc_Ws, *enc_bs, *dec_Ws, *dec_bs, cb0, cb1, cb2)

    rq_loss = loss[0, 0] * jnp.float32((1.0 + BETA) / (3.0 * N * D_Z))
    return (out, rq_loss, idx)


# speedup vs baseline: 2.7046x; 2.7046x over previous
import functools

import jax
import jax.numpy as jnp
from jax import lax
from jax.experimental import pallas as pl
from jax.experimental.pallas import tpu as pltpu

N = 4096
D_IN = 768
D_Z = 32
K = 8192
BR = 256
BETA = 0.25


def _dot_nt(a, b):
    return lax.dot_general(a, b, (((1,), (1,)), ((), ())),
                           preferred_element_type=jnp.float32)


def _onehot_gather(onehot_bf16, b):
    hi = b.astype(jnp.bfloat16)
    r1 = b - hi.astype(jnp.float32)
    mid = r1.astype(jnp.bfloat16)
    lo = (r1 - mid.astype(jnp.float32)).astype(jnp.bfloat16)
    dn = (((1,), (0,)), ((), ()))
    acc = lax.dot_general(onehot_bf16, hi, dn,
                          preferred_element_type=jnp.float32)
    acc = acc + lax.dot_general(onehot_bf16, mid, dn,
                                preferred_element_type=jnp.float32)
    return acc + lax.dot_general(onehot_bf16, lo, dn,
                                 preferred_element_type=jnp.float32)


def _body(x_ref,
          eW0, eW1, eW2, eW3, eb0, eb1, eb2, eb3,
          dW0, dW1, dW2, dW3, db0, db1, db2, db3,
          cb0_ref, cb1_ref, cb2_ref,
          out_ref, idx_ref, loss_ref):
    h = x_ref[...]
    enc = ((eW0, eb0), (eW1, eb1), (eW2, eb2), (eW3, eb3))
    for i, (w, b) in enumerate(enc):
        h = _dot_nt(h, w[...]) + b[...]
        if i != 3:
            h = jnp.maximum(h, 0.0)
    z = h

    res = z
    xq_acc = jnp.zeros_like(z)
    total = jnp.float32(0.0)
    idx_cols = []
    for cb_ref in (cb0_ref, cb1_ref, cb2_ref):
        cb = cb_ref[...]
        zsq = jnp.sum(res * res, axis=1, keepdims=True)
        cbsq = jnp.sum(cb * cb, axis=1)[None, :]
        d = (zsq + cbsq) - 2.0 * _dot_nt(res, cb)
        dmin = jnp.min(d, axis=1, keepdims=True)
        iota = lax.broadcasted_iota(jnp.int32, (BR, K), 1)
        idx = jnp.min(jnp.where(d == dmin, iota, K), axis=1)
        onehot = (iota == idx[:, None]).astype(jnp.bfloat16)
        xq = _onehot_gather(onehot, cb)
        diff = xq - res
        total = total + jnp.sum(diff * diff)
        res = res - xq
        xq_acc = xq_acc + xq
        idx_cols.append(idx[:, None])
    idx_ref[...] = jnp.concatenate(idx_cols, axis=1)

    h = xq_acc
    dec = ((dW0, db0), (dW1, db1), (dW2, db2), (dW3, db3))
    for i, (w, b) in enumerate(dec):
        h = _dot_nt(h, w[...]) + b[...]
        if i != 3:
            h = jnp.maximum(h, 0.0)
    out_ref[...] = h

    @pl.when(pl.program_id(0) == 0)
    def _():
        loss_ref[0, 0] = 0.0

    loss_ref[0, 0] += total


def kernel(x, enc_W0, enc_b0, enc_W1, enc_b1, enc_W2, enc_b2, enc_W3,
           enc_b3, dec_W0, dec_b0, dec_W1, dec_b1, dec_W2, dec_b2, dec_W3,
           dec_b3, cb0, cb1, cb2):
    enc_Ws = (enc_W0, enc_W1, enc_W2, enc_W3)
    enc_bs = tuple(b[None, :] for b in (enc_b0, enc_b1, enc_b2, enc_b3))
    dec_Ws = (dec_W0, dec_W1, dec_W2, dec_W3)
    dec_bs = tuple(b[None, :] for b in (dec_b0, dec_b1, dec_b2, dec_b3))

    full = lambda s: pl.BlockSpec(s, lambda i: (0,) * len(s))
    in_specs = (
        [pl.BlockSpec((BR, D_IN), lambda i: (i, 0))]
        + [full(w.shape) for w in enc_Ws]
        + [full(b.shape) for b in enc_bs]
        + [full(w.shape) for w in dec_Ws]
        + [full(b.shape) for b in dec_bs]
        + [full((K, D_Z))] * 3
    )
    out, idx, loss = pl.pallas_call(
        _body,
        grid=(N // BR,),
        in_specs=in_specs,
        out_specs=(
            pl.BlockSpec((BR, D_IN), lambda i: (i, 0)),
            pl.BlockSpec((BR, 3), lambda i: (i, 0)),
            pl.BlockSpec(memory_space=pltpu.SMEM),
        ),
        out_shape=(
            jax.ShapeDtypeStruct((N, D_IN), jnp.float32),
            jax.ShapeDtypeStruct((N, 3), jnp.int32),
            jax.ShapeDtypeStruct((1, 1), jnp.float32),
        ),
    )(x, *enc_Ws, *enc_bs, *dec_Ws, *dec_bs, cb0, cb1, cb2)

    rq_loss = loss[0, 0] * jnp.float32((1.0 + BETA) / (3.0 * N * D_Z))
    return (out, rq_loss, idx)
